# Initial kernel scaffold; baseline (speedup 1.0000x reference)
import jax
import jax.numpy as jnp
from jax.experimental import pallas as pl


def kernel(snip_features, topic_embedding, W1, b1, gamma1, beta1, Wt, bt, gammat, betat, Wg1, bg1, Wg2, bg2):
    raise NotImplementedError("write your pallas kernel here")


# speedup vs baseline: 14.4485x; 14.4485x over previous
import jax
import jax.numpy as jnp
from jax import lax
from jax.experimental import pallas as pl

B, C, T = 4, 256, 2048
TT = 128
K = 6
F32 = jnp.float32


def _prep_body(x_ref, D_ref, te_ref, Wtd_ref, P_ref,
               stats_ref, tf_ref, t2_ref):
    b = pl.program_id(0)
    xb = x_ref[0]
    zcol = jnp.zeros((C, 1), F32)
    xpad = jnp.concatenate([zcol, xb, zcol], axis=1)
    acc = None
    for s in range(3):
        d = lax.dot_general(D_ref[s], xpad[:, s:s + T],
                            (((1,), (0,)), ((), ())), preferred_element_type=F32)
        acc = d if acc is None else acc + d
    y = acc + P_ref[2][:, None]

    @pl.when(b == 0)
    def _():
        stats_ref[...] = jnp.zeros_like(stats_ref)

    stats_ref[0, :] += jnp.sum(y, axis=1)
    stats_ref[1, :] += jnp.sum(y * y, axis=1)

    te = te_ref[...]
    yts = []
    ts1 = jnp.zeros((C,), F32)
    ts2 = jnp.zeros((C,), F32)
    for bb in range(B):
        yt = lax.dot_general(Wtd_ref[...], te[bb],
                             (((1,), (1,)), ((), ())), preferred_element_type=F32)
        yt = yt + P_ref[5][:, None]
        yts.append(yt)
        ts1 = ts1 + jnp.sum(yt, axis=1)
        ts2 = ts2 + jnp.sum(yt * yt, axis=1)
    nmt = float(B * TT)
    meant = ts1 / nmt
    vart = ts2 / nmt - meant * meant
    scalet = P_ref[6] * lax.rsqrt(vart + 1e-5)
    shiftt = P_ref[7] - meant * scalet
    for bb in range(B):
        tf = jnp.maximum(yts[bb] * scalet[:, None] + shiftt[:, None], 0.0)
        tf_ref[bb] = tf
        t2_ref[bb] = jnp.broadcast_to(jnp.sum(tf * tf, axis=0)[None, :], (8, TT))


def _main_body(x_ref, D_ref, stats_ref, P_ref, A1_ref, A2_ref, Wd1_ref, Wd2_ref,
               tf_ref, t2_ref, out_ref):
    xb = x_ref[0]
    zcol = jnp.zeros((C, 1), F32)
    xpad = jnp.concatenate([zcol, xb, zcol], axis=1)
    acc = None
    for s in range(3):
        d = lax.dot_general(D_ref[s], xpad[:, s:s + T],
                            (((1,), (0,)), ((), ())), preferred_element_type=F32)
        acc = d if acc is None else acc + d
    y = acc + P_ref[2][:, None]
    n = float(B * T)
    mean = stats_ref[0] / n
    var = stats_ref[1] / n - mean * mean
    scale = P_ref[0] * lax.rsqrt(var + 1e-5)
    shift = P_ref[1] - mean * scale
    xn = jnp.maximum(y * scale[:, None] + shift[:, None], 0.0)

    tf = tf_ref[0]
    t2 = t2_ref[0, 0, :]
    iota = lax.broadcasted_iota(jnp.int32, (T, TT), 1)

    def gcn(xc, A, Wd, bg):
        cross = lax.dot_general(xc, tf, (((0,), (0,)), ((), ())),
                                preferred_element_type=F32)
        x2 = jnp.sum(xc * xc, axis=0)
        scores = 2.0 * cross - x2[:, None] - t2[None, :]
        agg = jnp.full((C // 2, T), -jnp.inf, F32)
        for _ in range(K):
            m = jnp.max(scores, axis=1, keepdims=True)
            cand = jnp.where(scores >= m, iota, TT)
            sel = jnp.min(cand, axis=1, keepdims=True)
            ohm = iota == sel
            oh = ohm.astype(F32)
            nbr = lax.dot_general(tf, oh, (((1,), (1,)), ((), ())),
                                  preferred_element_type=F32,
                                  precision=lax.Precision.HIGHEST)
            v = lax.dot_general(Wd, nbr - xc, (((1,), (0,)), ((), ())),
                                preferred_element_type=F32)
            agg = jnp.maximum(agg, v)
            scores = jnp.where(ohm, -jnp.inf, scores)
        lo = lax.dot_general(A, xc, (((1,), (0,)), ((), ())),
                             preferred_element_type=F32) + bg[:128][:, None]
        hi = agg + bg[128:][:, None]
        return jnp.maximum(xc + jnp.concatenate([lo, hi], axis=0), 0.0)

    x1 = gcn(xn, A1_ref[...], Wd1_ref[...], P_ref[3])
    xg = gcn(x1, A2_ref[...], Wd2_ref[...], P_ref[4])
    out_ref[0] = xg + xb


def _const(shape):
    nd = len(shape)
    return pl.BlockSpec(shape, lambda b: (0,) * nd)


def kernel(snip_features, topic_embedding, W1, b1, gamma1, beta1,
           Wt, bt, gammat, betat, Wg1, bg1, Wg2, bg2):
    eye4 = jnp.eye(4, dtype=F32)
    eye16 = jnp.eye(16, dtype=F32)
    W1r = W1.reshape(4, 64, 64, 3)
    D = jnp.einsum('gois,gh->sgohi', W1r, eye4).reshape(3, C, C)
    Wtd = jnp.einsum('goi,gh->gohi', Wt[:, :, 0].reshape(4, 64, 4), eye4).reshape(C, 16)
    A1 = jnp.einsum('gdc,gh->gdhc', Wg1[:16], eye16).reshape(C // 2, C)
    Wd1 = jnp.einsum('gdc,gh->gdhc', Wg1[16:], eye16).reshape(C // 2, C)
    A2 = jnp.einsum('gdc,gh->gdhc', Wg2[:16], eye16).reshape(C // 2, C)
    Wd2 = jnp.einsum('gdc,gh->gdhc', Wg2[16:], eye16).reshape(C // 2, C)
    P = jnp.stack([gamma1, beta1, b1, bg1, bg2, bt, gammat, betat])

    stats, tfa, t2 = pl.pallas_call(
        _prep_body,
        grid=(B,),
        in_specs=[
            pl.BlockSpec((1, C, T), lambda b: (b, 0, 0)),
            _const((3, C, C)),
            _const((B, TT, 16)),
            _const((C, 16)),
            _const((8, C)),
        ],
        out_specs=[
            _const((8, C)),
            _const((B, C, TT)),
            _const((B, 8, TT)),
        ],
        out_shape=[
            jax.ShapeDtypeStruct((8, C), F32),
            jax.ShapeDtypeStruct((B, C, TT), F32),
            jax.ShapeDtypeStruct((B, 8, TT), F32),
        ],
    )(snip_features, D, topic_embedding, Wtd, P)

    out = pl.pallas_call(
        _main_body,
        grid=(B,),
        in_specs=[
            pl.BlockSpec((1, C, T), lambda b: (b, 0, 0)),
            _const((3, C, C)),
            _const((8, C)),
            _const((8, C)),
            _const((C // 2, C)),
            _const((C // 2, C)),
            _const((C // 2, C)),
            _const((C // 2, C)),
            pl.BlockSpec((1, C, TT), lambda b: (b, 0, 0)),
            pl.BlockSpec((1, 8, TT), lambda b: (b, 0, 0)),
        ],
        out_specs=pl.BlockSpec((1, C, T), lambda b: (b, 0, 0)),
        out_shape=jax.ShapeDtypeStruct((B, C, T), F32),
    )(snip_features, D, stats, P, A1, A2, Wd1, Wd2, tfa, t2)
    return out
